# speedup vs baseline: 3.8003x; 3.8003x over previous
import functools

import jax
import jax.numpy as jnp
from jax import lax
from jax.experimental import pallas as pl
from jax.experimental.pallas import tpu as pltpu
from jax.experimental.pallas import tpu_sc as plsc

D = 512
L = 200
B = 1024
NC = 2
NS = 16
NW = NC * NS
ROWS = L * B
RPW = ROWS // NW
C = 80
NCHUNK = RPW // C
NBUF = 2

_mesh = plsc.VectorSubcoreMesh(core_axis_name="c", subcore_axis_name="s")


@functools.partial(
    pl.kernel,
    mesh=_mesh,
    out_type=jax.ShapeDtypeStruct((ROWS, D), jnp.float32),
    scratch_types=[
        pltpu.VMEM((NCHUNK, C), jnp.int32),
        pltpu.VMEM((NBUF, C, D), jnp.float32),
        pltpu.SemaphoreType.DMA,
        pltpu.SemaphoreType.DMA,
    ],
)
def _pe_gather(idx_hbm, table_hbm, out_hbm, idx_v, buf_v, gsem, osem):
    wid = lax.axis_index("s") * NC + lax.axis_index("c")
    base = wid * RPW

    pltpu.sync_copy(idx_hbm.at[wid], idx_v)

    for b in range(NBUF):
        pltpu.async_copy(table_hbm.at[idx_v.at[b]], buf_v.at[b], gsem)

    def body(i, _):
        g = i * NBUF
        for b in range(NBUF):
            j = g + b
            buf = buf_v.at[b]
            dst = out_hbm.at[pl.ds(base + j * C, C)]
            pltpu.make_async_copy(table_hbm.at[idx_v.at[j]], buf, gsem).wait()
            pltpu.async_copy(buf, dst, osem)
            pltpu.make_async_copy(buf, dst, osem).wait()

            @pl.when(j + NBUF < NCHUNK)
            def _():
                pltpu.async_copy(
                    table_hbm.at[idx_v.at[j + NBUF]], buf, gsem
                )

        return ()

    lax.fori_loop(0, NCHUNK // NBUF, body, ())


def kernel(x, pe):
    idx = x.T.reshape(NW, NCHUNK, C)
    out = _pe_gather(idx, pe)
    return out.reshape(L, B, D)
